# baseline (speedup 1.0000x reference)
import functools

import jax
import jax.numpy as jnp
from jax import lax
from jax.experimental import pallas as pl
from jax.experimental.pallas import tpu as pltpu
from jax.experimental.pallas import tpu_sc as plsc

_NC = 2
_NS = 16
_NW = _NC * _NS
_L = 16

_GW = 80


def _build(n_rows, seqlen, d_model):
  assert n_rows % (_NW * seqlen) == 0
  rows_w = n_rows // _NW
  seqs_chunk = 2
  rows_chunk = seqs_chunk * seqlen
  n_chunks = rows_w // rows_chunk
  n_gathers = rows_chunk // _GW
  vpr = d_model // _L

  mesh = plsc.VectorSubcoreMesh(
      core_axis_name="c", subcore_axis_name="s",
      num_cores=_NC, num_subcores=_NS)

  @functools.partial(
      pl.kernel,
      out_type=(
          jax.ShapeDtypeStruct((n_rows, d_model), jnp.float32),
          jax.ShapeDtypeStruct((n_rows,), jnp.float32),
      ),
      mesh=mesh,
      scratch_types=[
          pltpu.VMEM((rows_chunk,), jnp.int32),
          pltpu.VMEM((n_gathers, _GW), jnp.int32),
          pltpu.VMEM((rows_chunk, 2 * d_model), jnp.float32),
          pltpu.VMEM((rows_chunk, d_model), jnp.float32),
          pltpu.VMEM((seqlen // 2, 2 * d_model), jnp.float32),
          pltpu.VMEM((rows_chunk,), jnp.float32),
          pltpu.SemaphoreType.DMA,
      ],
  )
  def embed(idxf_hbm, tab2_hbm, pos2_hbm, out_hbm, mask_hbm,
            idxf_v, pidx_v, rows_v, out_v, pos_v, mask_v, sem):
    wid = lax.axis_index("s") * _NC + lax.axis_index("c")
    base = wid * rows_w
    pltpu.sync_copy(pos2_hbm, pos_v)

    def chunk_body(c, carry):
      rb = base + c * rows_chunk
      pltpu.sync_copy(idxf_hbm.at[pl.ds(rb, rows_chunk)], idxf_v)
      for j in range(rows_chunk // _L):
        iv = idxf_v[pl.ds(j * _L, _L)]
        g, l = divmod(j, _GW // _L)
        pidx_v[g, pl.ds(l * _L, _L)] = iv >> 1
      copies = [
          pltpu.async_copy(tab2_hbm.at[pidx_v.at[g]],
                           rows_v.at[pl.ds(g * _GW, _GW)], sem)
          for g in range(n_gathers)
      ]
      for j in range(rows_chunk // _L):
        ids = idxf_v[pl.ds(j * _L, _L)]
        mask_v[pl.ds(j * _L, _L)] = jnp.where(
            ids == 0, jnp.float32(-jnp.inf), jnp.float32(0.0))
      for cp in copies:
        cp.wait()

      def pos_body(m, pc):
        hv = (idxf_v[pl.ds(m * _L, _L)] & 1) * d_model
        for lane in range(_L):
          r = m * _L + lane
          p = lax.select(r >= seqlen, r - seqlen, r)
          q = p >> 1
          par = p & 1
          h = hv[lane]
          for v in range(vpr):
            pv = pos_v[q, pl.ds(par * d_model + v * _L, _L)]
            tv = rows_v[r, pl.ds(h + v * _L, _L)]
            out_v[r, pl.ds(v * _L, _L)] = tv + pv
        return pc
      lax.fori_loop(0, rows_chunk // _L, pos_body, 0)

      pltpu.sync_copy(out_v, out_hbm.at[pl.ds(rb, rows_chunk)])
      pltpu.sync_copy(mask_v, mask_hbm.at[pl.ds(rb, rows_chunk)])
      return carry

    lax.fori_loop(0, n_chunks, chunk_body, 0)

  return embed


def kernel(tgt, embedding_tgt, embedding_pos):
  batch, seqlen = tgt.shape
  d_model = embedding_tgt.shape[1]
  n_rows = batch * seqlen
  idx_flat = tgt.reshape(n_rows)
  tab2 = embedding_tgt.reshape(-1, 2 * d_model)
  pos2 = embedding_pos.reshape(-1, 2 * d_model)
  embed = _build(n_rows, seqlen, d_model)
  out, mask = embed(idx_flat, tab2, pos2)
  return out.reshape(batch, seqlen, d_model), mask.reshape(batch, seqlen)


# speedup vs baseline: 1.4419x; 1.4419x over previous
import functools

import jax
import jax.numpy as jnp
from jax import lax
from jax.experimental import pallas as pl
from jax.experimental.pallas import tpu as pltpu
from jax.experimental.pallas import tpu_sc as plsc

_NC = 2
_NS = 16
_NW = _NC * _NS
_L = 16

_GW = 80


def _build(n_rows, seqlen, d_model):
  assert n_rows % (_NW * seqlen) == 0
  rows_w = n_rows // _NW
  seqs_chunk = 2
  rows_chunk = seqs_chunk * seqlen
  n_chunks = rows_w // rows_chunk
  n_gathers = rows_chunk // _GW
  vpr = d_model // _L

  mesh = plsc.VectorSubcoreMesh(
      core_axis_name="c", subcore_axis_name="s",
      num_cores=_NC, num_subcores=_NS)

  @functools.partial(
      pl.kernel,
      out_type=(
          jax.ShapeDtypeStruct((n_rows, d_model), jnp.float32),
          jax.ShapeDtypeStruct((n_rows,), jnp.float32),
      ),
      mesh=mesh,
      scratch_types=[
          pltpu.VMEM((rows_chunk,), jnp.int32),
          pltpu.VMEM((n_gathers, _GW), jnp.int32),
          pltpu.VMEM((rows_chunk, 2 * d_model), jnp.float32),
          pltpu.VMEM((rows_chunk, d_model), jnp.float32),
          pltpu.VMEM((seqlen, d_model), jnp.float32),
          pltpu.VMEM((rows_chunk,), jnp.float32),
          pltpu.SemaphoreType.DMA,
      ],
  )
  def embed(idxf_hbm, tab2_hbm, pos2_hbm, out_hbm, mask_hbm,
            idxf_v, pidx_v, rows_v, out_v, pos_v, mask_v, sem):
    wid = lax.axis_index("s") * _NC + lax.axis_index("c")
    base = wid * rows_w
    pltpu.sync_copy(pos2_hbm, pos_v)

    def chunk_body(c, carry):
      rb = base + c * rows_chunk
      pltpu.sync_copy(idxf_hbm.at[pl.ds(rb, rows_chunk)], idxf_v)
      for j in range(rows_chunk // _L):
        iv = idxf_v[pl.ds(j * _L, _L)]
        g, l = divmod(j, _GW // _L)
        pidx_v[g, pl.ds(l * _L, _L)] = iv
      copies = [
          pltpu.async_copy(tab2_hbm.at[pidx_v.at[g]],
                           rows_v.at[pl.ds(g * _GW, _GW)], sem)
          for g in range(n_gathers)
      ]
      for j in range(rows_chunk // _L):
        ids = idxf_v[pl.ds(j * _L, _L)]
        mask_v[pl.ds(j * _L, _L)] = jnp.where(
            ids == 0, jnp.float32(-jnp.inf), jnp.float32(0.0))
      for cp in copies:
        cp.wait()

      def pos_body(p, pc):
        for v in range(vpr):
          sl = pl.ds(v * _L, _L)
          pv = pos_v[p, sl]
          for s in range(seqs_chunk):
            r = s * seqlen + p
            out_v[r, sl] = rows_v[r, sl] + pv
        return pc
      lax.fori_loop(0, seqlen, pos_body, 0)

      pltpu.sync_copy(out_v, out_hbm.at[pl.ds(rb, rows_chunk)])
      pltpu.sync_copy(mask_v, mask_hbm.at[pl.ds(rb, rows_chunk)])
      return carry

    lax.fori_loop(0, n_chunks, chunk_body, 0)

  return embed


def kernel(tgt, embedding_tgt, embedding_pos):
  batch, seqlen = tgt.shape
  d_model = embedding_tgt.shape[1]
  n_rows = batch * seqlen
  idx_flat = tgt.reshape(n_rows)
  tab2 = jnp.pad(embedding_tgt, ((0, 0), (0, d_model)))
  embed = _build(n_rows, seqlen, d_model)
  out, mask = embed(idx_flat, tab2, embedding_pos)
  return out.reshape(batch, seqlen, d_model), mask.reshape(batch, seqlen)


# speedup vs baseline: 1.5718x; 1.0900x over previous
import functools

import jax
import jax.numpy as jnp
from jax import lax
from jax.experimental import pallas as pl
from jax.experimental.pallas import tpu as pltpu
from jax.experimental.pallas import tpu_sc as plsc

_NC = 2
_NS = 16
_NW = _NC * _NS
_L = 16

_GW = 80


def _build(n_rows, seqlen, d_model):
  assert n_rows % (_NW * seqlen) == 0
  rows_w = n_rows // _NW
  seqs_chunk = 2
  rows_chunk = seqs_chunk * seqlen
  n_chunks = rows_w // rows_chunk
  n_gathers = rows_chunk // _GW
  vpr = d_model // _L

  mesh = plsc.VectorSubcoreMesh(
      core_axis_name="c", subcore_axis_name="s",
      num_cores=_NC, num_subcores=_NS)

  @functools.partial(
      pl.kernel,
      out_type=jax.ShapeDtypeStruct((n_rows, d_model), jnp.float32),
      mesh=mesh,
      scratch_types=(
          [
              pltpu.VMEM((rows_chunk,), jnp.int32),
              pltpu.VMEM((n_gathers, _GW), jnp.int32),
              pltpu.VMEM((rows_chunk, 2 * d_model), jnp.float32),
              pltpu.VMEM((rows_chunk, d_model), jnp.float32),
              pltpu.VMEM((seqlen, d_model), jnp.float32),
          ]
          + [pltpu.SemaphoreType.DMA] * n_gathers
      ),
  )
  def embed(idxf_hbm, tab2_hbm, pos_hbm, out_hbm, *scr):
    idx_v, pidx_v, rows_v, out_v, pos_v = scr[0:5]
    sg = scr[5:5 + n_gathers]
    wid = lax.axis_index("s") * _NC + lax.axis_index("c")
    base = wid * rows_w
    pltpu.sync_copy(pos_hbm, pos_v)

    def stage_pidx():
      for j in range(rows_chunk // _L):
        iv = idx_v[pl.ds(j * _L, _L)]
        g, l = divmod(j, _GW // _L)
        pidx_v[g, pl.ds(l * _L, _L)] = iv

    def fire_gathers():
      for g in range(n_gathers):
        pltpu.async_copy(tab2_hbm.at[pidx_v.at[g]],
                         rows_v.at[pl.ds(g * _GW, _GW)], sg[g])

    def drain(sem, src, dst):
      pltpu.make_async_copy(src, dst, sem).wait()

    def add_block(g):
      spans = []
      r0 = g * _GW
      p0 = r0 % seqlen
      if p0 + _GW <= seqlen:
        spans.append((r0, p0, _GW))
      else:
        first = seqlen - p0
        spans.append((r0, p0, first))
        spans.append((r0 + first, 0, _GW - first))
      for (rb0, pb0, ln) in spans:
        def body(i, pc):
          r = rb0 + i
          p = pb0 + i
          for v in range(vpr):
            sl = pl.ds(v * _L, _L)
            out_v[r, sl] = rows_v[r, sl] + pos_v[p, sl]
          return pc
        lax.fori_loop(0, ln, body, 0)

    pltpu.sync_copy(idxf_hbm.at[pl.ds(base, rows_chunk)], idx_v)
    stage_pidx()
    fire_gathers()

    def chunk_body(c, carry):
      rb = base + c * rows_chunk
      for g in range(n_gathers):
        drain(sg[g], tab2_hbm.at[pl.ds(0, _GW)],
              rows_v.at[pl.ds(g * _GW, _GW)])
        add_block(g)

      @pl.when(c + 1 < n_chunks)
      def _():
        pltpu.sync_copy(idxf_hbm.at[pl.ds(rb + rows_chunk, rows_chunk)],
                        idx_v)
        stage_pidx()
        fire_gathers()

      pltpu.sync_copy(out_v, out_hbm.at[pl.ds(rb, rows_chunk)])
      return carry

    lax.fori_loop(0, n_chunks, chunk_body, 0)

  return embed


def kernel(tgt, embedding_tgt, embedding_pos):
  batch, seqlen = tgt.shape
  d_model = embedding_tgt.shape[1]
  n_rows = batch * seqlen
  idx_flat = tgt.reshape(n_rows)
  tab2 = jnp.pad(embedding_tgt, ((0, 0), (0, d_model)))
  embed = _build(n_rows, seqlen, d_model)
  out = embed(idx_flat, tab2, embedding_pos)
  mask = jnp.where(tgt == 0, -jnp.inf, 0.0).astype(jnp.float32)
  return out.reshape(batch, seqlen, d_model), mask


# speedup vs baseline: 1.6482x; 1.0486x over previous
import functools

import jax
import jax.numpy as jnp
from jax import lax
from jax.experimental import pallas as pl
from jax.experimental.pallas import tpu as pltpu
from jax.experimental.pallas import tpu_sc as plsc

_NC = 2
_NS = 16
_NW = _NC * _NS
_L = 16

_GW = 80


def _build(n_rows, seqlen, d_model):
  assert n_rows % (_NW * seqlen) == 0
  rows_w = n_rows // _NW
  seqs_chunk = 2
  rows_chunk = seqs_chunk * seqlen
  n_chunks = rows_w // rows_chunk
  n_gathers = rows_chunk // _GW
  vpr = d_model // _L

  mesh = plsc.VectorSubcoreMesh(
      core_axis_name="c", subcore_axis_name="s",
      num_cores=_NC, num_subcores=_NS)

  @functools.partial(
      pl.kernel,
      out_type=jax.ShapeDtypeStruct((n_rows, d_model), jnp.float32),
      mesh=mesh,
      scratch_types=(
          [
              pltpu.VMEM((rows_chunk,), jnp.int32),
              pltpu.VMEM((rows_chunk,), jnp.int32),
              pltpu.VMEM((n_gathers, _GW), jnp.int32),
              pltpu.VMEM((n_gathers, _GW), jnp.int32),
              pltpu.VMEM((rows_chunk, 2 * d_model), jnp.float32),
              pltpu.VMEM((rows_chunk, d_model), jnp.float32),
              pltpu.VMEM((seqlen, d_model), jnp.float32),
          ]
          + [pltpu.SemaphoreType.DMA] * 2
          + [pltpu.SemaphoreType.DMA] * n_gathers
      ),
  )
  def embed(idxf_hbm, tab2_hbm, pos_hbm, out_hbm, *scr):
    idx_v = scr[0:2]
    pidx_v = scr[2:4]
    rows_v, out_v, pos_v = scr[4:7]
    si = scr[7:9]
    sg = scr[9:9 + n_gathers]
    wid = lax.axis_index("s") * _NC + lax.axis_index("c")
    base = wid * rows_w
    pltpu.sync_copy(pos_hbm, pos_v)

    def stage_pidx(b):
      for j in range(rows_chunk // _L):
        iv = idx_v[b][pl.ds(j * _L, _L)]
        g, l = divmod(j, _GW // _L)
        pidx_v[b][g, pl.ds(l * _L, _L)] = iv

    def fire_gather(b, g):
      pltpu.async_copy(tab2_hbm.at[pidx_v[b].at[g]],
                       rows_v.at[pl.ds(g * _GW, _GW)], sg[g])

    def drain(sem, src, dst):
      pltpu.make_async_copy(src, dst, sem).wait()

    def add_block(g):
      spans = []
      r0 = g * _GW
      p0 = r0 % seqlen
      if p0 + _GW <= seqlen:
        spans.append((r0, p0, _GW))
      else:
        first = seqlen - p0
        spans.append((r0, p0, first))
        spans.append((r0 + first, 0, _GW - first))
      for (rb0, pb0, ln) in spans:
        def body(i, pc):
          r = rb0 + i
          p = pb0 + i
          for v in range(vpr):
            sl = pl.ds(v * _L, _L)
            out_v[r, sl] = rows_v[r, sl] + pos_v[p, sl]
          return pc
        lax.fori_loop(0, ln, body, 0)

    pltpu.sync_copy(idxf_hbm.at[pl.ds(base, rows_chunk)], idx_v[0])
    stage_pidx(0)
    for g in range(n_gathers):
      fire_gather(0, g)
    pltpu.async_copy(idxf_hbm.at[pl.ds(base + rows_chunk, rows_chunk)],
                     idx_v[1], si[1])

    def pair_body(c2, carry):
      for k in range(2):
        c = 2 * c2 + k
        b = k
        nb = 1 - k
        rb = base + c * rows_chunk

        @pl.when(c + 1 < n_chunks)
        def _():
          drain(si[nb], idxf_hbm.at[pl.ds(base, rows_chunk)], idx_v[nb])
          stage_pidx(nb)

        @pl.when(c + 2 < n_chunks)
        def _():
          pltpu.async_copy(
              idxf_hbm.at[pl.ds(rb + 2 * rows_chunk, rows_chunk)],
              idx_v[b], si[b])

        for g in range(n_gathers):
          drain(sg[g], tab2_hbm.at[pl.ds(0, _GW)],
                rows_v.at[pl.ds(g * _GW, _GW)])
          add_block(g)
          @pl.when(c + 1 < n_chunks)
          def _():
            fire_gather(nb, g)

        pltpu.sync_copy(out_v, out_hbm.at[pl.ds(rb, rows_chunk)])
      return carry

    lax.fori_loop(0, n_chunks // 2, pair_body, 0)

  return embed


def kernel(tgt, embedding_tgt, embedding_pos):
  batch, seqlen = tgt.shape
  d_model = embedding_tgt.shape[1]
  n_rows = batch * seqlen
  idx_flat = tgt.reshape(n_rows)
  tab2 = jnp.pad(embedding_tgt, ((0, 0), (0, d_model)))
  embed = _build(n_rows, seqlen, d_model)
  out = embed(idx_flat, tab2, embedding_pos)
  mask = jnp.where(tgt == 0, -jnp.inf, 0.0).astype(jnp.float32)
  return out.reshape(batch, seqlen, d_model), mask
